# baseline (speedup 1.0000x reference)
import functools

import jax
import jax.numpy as jnp
from jax import lax
from jax.experimental import pallas as pl
from jax.experimental.pallas import tpu as pltpu
from jax.experimental.pallas import tpu_sc as plsc

N = 10000
D = 128
G = 64
N_PAD = 10240
BLK = 512
NUM_BLK = N_PAD // BLK
NC = 2
NS = 16
NW = NC * NS
K = 4
SLAB = N_PAD // NS
DEGW = 128


def _sc_mesh():
    return plsc.VectorSubcoreMesh(core_axis_name="c", subcore_axis_name="s")


NBUF = 4
AGW = 64
AGK = 8


def _make_agg(idx_rows):
    idx_rows2 = idx_rows * (128 // AGW)
    rows_per_tile = idx_rows2 // NW
    groups = rows_per_tile // AGK

    @functools.partial(
        pl.kernel,
        out_type=jax.ShapeDtypeStruct((NC, N_PAD, D), jnp.float32),
        mesh=_sc_mesh(),
        scratch_types=[
            pltpu.VMEM((AGK, AGW), jnp.int32),
            pltpu.VMEM((AGK, AGW), jnp.int32),
            pltpu.VMEM((AGW, D), jnp.float32),
            pltpu.VMEM((AGW, D), jnp.float32),
            pltpu.VMEM((AGW, D), jnp.float32),
            pltpu.VMEM((AGW, D), jnp.float32),
            pltpu.VMEM_SHARED((N_PAD, D), jnp.float32),
            pltpu.SemaphoreType.DMA,
            pltpu.SemaphoreType.DMA,
            pltpu.SemaphoreType.DMA,
            pltpu.SemaphoreType.DMA,
            pltpu.SemaphoreType.DMA,
            pltpu.SemaphoreType.DMA,
            pltpu.SemaphoreType.DMA,
            pltpu.SemaphoreType.DMA,
        ],
    )
    def agg(src_hbm, dst_hbm, u_hbm, z_hbm, out_hbm, src_v, dst_v, rows0,
            rows1, rows2, rows3, acc, gsem0, gsem1, gsem2, gsem3, ssem0,
            ssem1, ssem2, ssem3):
        cid = lax.axis_index("c")
        sid = lax.axis_index("s")
        wid = sid * NC + cid
        shard = wid * rows_per_tile

        pltpu.sync_copy(z_hbm, rows0)
        for j in range(SLAB // AGW):
            pltpu.sync_copy(rows0, acc.at[pl.ds(sid * SLAB + j * AGW, AGW)])
        plsc.subcore_barrier()

        bufs = (rows0, rows1, rows2, rows3)
        gsems = (gsem0, gsem1, gsem2, gsem3)
        ssems = (ssem0, ssem1, ssem2, ssem3)

        def body(g, carry):
            base = shard + g * AGK
            pltpu.sync_copy(src_hbm.at[pl.ds(base, AGK)], src_v)
            pltpu.sync_copy(dst_hbm.at[pl.ds(base, AGK)], dst_v)
            gat = [None] * NBUF
            scat = [None] * NBUF
            for j in range(NBUF - 1):
                gat[j] = pltpu.async_copy(
                    u_hbm.at[src_v.at[j]], bufs[j], gsems[j])
            for j in range(AGK):
                b = j % NBUF
                if j + NBUF - 1 < AGK:
                    b3 = (j + NBUF - 1) % NBUF
                    if scat[b3] is not None:
                        scat[b3].wait()
                    gat[b3] = pltpu.async_copy(
                        u_hbm.at[src_v.at[j + NBUF - 1]], bufs[b3],
                        gsems[b3])
                gat[b].wait()
                scat[b] = pltpu.async_copy(
                    bufs[b], acc.at[dst_v.at[j]], ssems[b], add=True)
            for b in range(NBUF):
                scat[b].wait()
            return carry

        lax.fori_loop(0, groups, body, 0)

        plsc.subcore_barrier()
        for j in range(SLAB // AGW):
            sl = pl.ds(sid * SLAB + j * AGW, AGW)
            pltpu.sync_copy(acc.at[sl], rows0)
            pltpu.sync_copy(rows0, out_hbm.at[cid, sl])

    return agg


def _make_deg(idx_rows):
    idx_rows2 = idx_rows * (128 // AGW)
    rows_per_tile = idx_rows2 // NW
    dgk = 2 * AGK
    groups = rows_per_tile // dgk

    @functools.partial(
        pl.kernel,
        out_type=jax.ShapeDtypeStruct((NC, N_PAD, DEGW), jnp.float32),
        mesh=_sc_mesh(),
        scratch_types=[
            pltpu.VMEM((dgk, AGW), jnp.int32),
            pltpu.VMEM((AGW, DEGW), jnp.float32),
            pltpu.VMEM_SHARED((N_PAD, DEGW), jnp.float32),
            pltpu.SemaphoreType.DMA,
        ],
    )
    def deg(dst_hbm, z_hbm, ones_hbm, out_hbm, dst_v, rows_v, acc, ssem):
        cid = lax.axis_index("c")
        sid = lax.axis_index("s")
        wid = sid * NC + cid

        pltpu.sync_copy(z_hbm, rows_v)
        for j in range(SLAB // AGW):
            pltpu.sync_copy(rows_v, acc.at[pl.ds(sid * SLAB + j * AGW, AGW)])
        pltpu.sync_copy(ones_hbm, rows_v)
        plsc.subcore_barrier()

        def body(g, carry):
            base = wid * rows_per_tile + g * dgk
            pltpu.sync_copy(dst_hbm.at[pl.ds(base, dgk)], dst_v)
            descs = [pltpu.async_copy(rows_v, acc.at[dst_v.at[j]], ssem,
                                      add=True)
                     for j in range(dgk)]
            for dsc in descs:
                dsc.wait()
            return carry

        lax.fori_loop(0, groups, body, 0)

        plsc.subcore_barrier()
        for j in range(SLAB // AGW):
            sl = pl.ds(sid * SLAB + j * AGW, AGW)
            pltpu.sync_copy(acc.at[sl], rows_v)
            pltpu.sync_copy(rows_v, out_hbm.at[cid, sl])

    return deg


def _rowmask(i):
    return i * BLK + lax.broadcasted_iota(jnp.int32, (BLK, 1), 0) < N


def _mm_body(x_ref, w_ref, o_ref):
    o_ref[...] = jnp.dot(x_ref[...], w_ref[...],
                         preferred_element_type=jnp.float32)


def _scale_body(dp_ref, wx_ref, u_ref, dinv_ref):
    i = pl.program_id(0)
    degsum = dp_ref[0, :, 0:1] + dp_ref[1, :, 0:1] + 1.0
    dinv = jnp.where(_rowmask(i), lax.rsqrt(degsum), 0.0)
    dinv_ref[...] = dinv
    u_ref[...] = wx_ref[...] * dinv


def _post_body(p_ref, u_ref, b_ref, dinv_ref, y_ref, s_ref, q_ref):
    i = pl.program_id(0)
    y = (p_ref[0] + p_ref[1] + u_ref[...]) * dinv_ref[...] + b_ref[...]
    y_ref[...] = y
    ym = jnp.where(_rowmask(i), y, 0.0)

    @pl.when(i == 0)
    def _():
        s_ref[...] = jnp.zeros_like(s_ref)
        q_ref[...] = jnp.zeros_like(q_ref)

    s_ref[0:1, :] = s_ref[0:1, :] + jnp.sum(ym, axis=0, keepdims=True)
    q_ref[0:1, :] = q_ref[0:1, :] + jnp.sum(ym * ym, axis=0, keepdims=True)


def _bn_relu(y, s_ref, q_ref, g_ref, be_ref):
    mu = s_ref[0:1, :] * (1.0 / N)
    var = q_ref[0:1, :] * (1.0 / N) - mu * mu
    scale = lax.rsqrt(var + 1e-5) * g_ref[...]
    return jnp.maximum((y - mu) * scale + be_ref[...], 0.0)


def _mid_body(y_ref, s_ref, q_ref, g_ref, be_ref, w_ref, dinv_ref, u_ref):
    h = _bn_relu(y_ref[...], s_ref, q_ref, g_ref, be_ref)
    u_ref[...] = (
        jnp.dot(h, w_ref[...], preferred_element_type=jnp.float32)
        * dinv_ref[...])


def _pool_body(y_ref, s_ref, q_ref, g_ref, be_ref, batch_ref, sums_ref,
               cnt_ref):
    i = pl.program_id(0)
    h = _bn_relu(y_ref[...], s_ref, q_ref, g_ref, be_ref)

    @pl.when(i == 0)
    def _():
        sums_ref[...] = jnp.zeros_like(sums_ref)
        cnt_ref[...] = jnp.zeros_like(cnt_ref)

    gids = lax.broadcasted_iota(jnp.int32, (G, 128), 0)
    sums = sums_ref[...]
    cnt = cnt_ref[...]
    for half in range(BLK // 128):
        b = batch_ref[0, half:half + 1, :]
        onehot = (gids == b).astype(jnp.float32)
        sums = sums + jnp.dot(onehot, h[half * 128:(half + 1) * 128, :],
                              preferred_element_type=jnp.float32)
        cnt = cnt + jnp.broadcast_to(
            jnp.sum(onehot, axis=1, keepdims=True), (G, 128))
    sums_ref[...] = sums
    cnt_ref[...] = cnt


def _heads_body(sums_ref, cnt_ref, wc_ref, bc_ref, wd1_ref, bd1_ref, wd2_ref,
                bd2_ref, cls_ref, dom_ref, feat_ref):
    cnt = jnp.maximum(cnt_ref[...], 1.0)
    feat = sums_ref[...] / cnt
    feat_ref[...] = feat
    cls_ref[...] = jnp.dot(
        feat, wc_ref[...], preferred_element_type=jnp.float32) + bc_ref[...]
    hd = jnp.maximum(
        jnp.dot(feat, wd1_ref[...], preferred_element_type=jnp.float32)
        + bd1_ref[...], 0.0)
    dom_ref[...] = jnp.dot(
        hd, wd2_ref[...], preferred_element_type=jnp.float32) + bd2_ref[...]


def _blk(i):
    return (i, 0)


def _const(i):
    return (0, 0)


_ROWSPEC = pl.BlockSpec((BLK, 128), _blk)
_COLSPEC = pl.BlockSpec((BLK, 1), _blk)
_FULLW = pl.BlockSpec((128, 128), _const)
_VEC = pl.BlockSpec((1, 128), _const)
_STAT = pl.BlockSpec((8, 128), _const)
_PSPEC = pl.BlockSpec((NC, BLK, 128), lambda i: (0, i, 0))


def _mm_call(xp, w):
    return pl.pallas_call(
        _mm_body,
        grid=(NUM_BLK,),
        in_specs=[_ROWSPEC, _FULLW],
        out_specs=_ROWSPEC,
        out_shape=jax.ShapeDtypeStruct((N_PAD, D), jnp.float32),
    )(xp, w)


def _scale_call(deg_p, wx):
    return pl.pallas_call(
        _scale_body,
        grid=(NUM_BLK,),
        in_specs=[pl.BlockSpec((NC, BLK, DEGW), lambda i: (0, i, 0)),
                  _ROWSPEC],
        out_specs=[_ROWSPEC, _COLSPEC],
        out_shape=[jax.ShapeDtypeStruct((N_PAD, D), jnp.float32),
                   jax.ShapeDtypeStruct((N_PAD, 1), jnp.float32)],
    )(deg_p, wx)


def _post_call(p, u, b, dinv):
    return pl.pallas_call(
        _post_body,
        grid=(NUM_BLK,),
        in_specs=[_PSPEC, _ROWSPEC, _VEC, _COLSPEC],
        out_specs=[_ROWSPEC, _STAT, _STAT],
        out_shape=[jax.ShapeDtypeStruct((N_PAD, D), jnp.float32),
                   jax.ShapeDtypeStruct((8, 128), jnp.float32),
                   jax.ShapeDtypeStruct((8, 128), jnp.float32)],
    )(p, u, b, dinv)


def _mid_call(y, s, q, g, be, w, dinv):
    return pl.pallas_call(
        _mid_body,
        grid=(NUM_BLK,),
        in_specs=[_ROWSPEC, _STAT, _STAT, _VEC, _VEC, _FULLW, _COLSPEC],
        out_specs=_ROWSPEC,
        out_shape=jax.ShapeDtypeStruct((N_PAD, D), jnp.float32),
    )(y, s, q, g, be, w, dinv)


def _pool_call(y, s, q, g, be, batch3d):
    return pl.pallas_call(
        _pool_body,
        grid=(NUM_BLK,),
        in_specs=[_ROWSPEC, _STAT, _STAT, _VEC, _VEC,
                  pl.BlockSpec((1, BLK // 128, 128), lambda i: (i, 0, 0))],
        out_specs=[pl.BlockSpec((G, 128), _const),
                   pl.BlockSpec((G, 128), _const)],
        out_shape=[jax.ShapeDtypeStruct((G, 128), jnp.float32),
                   jax.ShapeDtypeStruct((G, 128), jnp.float32)],
    )(y, s, q, g, be, batch3d)


def _heads_call(sums, cnt, wc, bc, wd1, bd1, wd2, bd2):
    return pl.pallas_call(
        _heads_body,
        out_shape=[jax.ShapeDtypeStruct((G, wc.shape[1]), jnp.float32),
                   jax.ShapeDtypeStruct((G, 2), jnp.float32),
                   jax.ShapeDtypeStruct((G, 128), jnp.float32)],
    )(sums, cnt, wc, bc, wd1, bd1, wd2, bd2)


def kernel(x, edge_index, batch, W0, b0, g0, be0, W1, b1, g1, be1,
           W2, b2, g2, be2, Wc, bc, Wd1, bd1, Wd2, bd2):
    e = edge_index.shape[1]
    gran = AGW * 2 * AGK * NW
    e_pad = -(-e // gran) * gran
    idx_rows = e_pad // 128
    npad = N_PAD - N

    src = edge_index[0].astype(jnp.int32)
    dst = edge_index[1].astype(jnp.int32)
    pad = (N + jnp.arange(e_pad - e, dtype=jnp.int32) % npad)
    srcp = jnp.concatenate([src, pad]).reshape(-1, AGW)
    dstp = jnp.concatenate([dst, pad]).reshape(-1, AGW)

    xp = jnp.zeros((N_PAD, D), jnp.float32).at[:N].set(x)
    batchp = jnp.concatenate(
        [batch.astype(jnp.int32),
         jnp.full((npad,), G + 63, jnp.int32)]).reshape(NUM_BLK, BLK // 128,
                                                        128)
    zeros = jnp.zeros((AGW, D), jnp.float32)
    zeros16 = jnp.zeros((AGW, DEGW), jnp.float32)
    ones16 = jnp.ones((AGW, DEGW), jnp.float32)

    b0r, g0r, be0r = b0.reshape(1, D), g0.reshape(1, D), be0.reshape(1, D)
    b1r, g1r, be1r = b1.reshape(1, D), g1.reshape(1, D), be1.reshape(1, D)
    b2r, g2r, be2r = b2.reshape(1, D), g2.reshape(1, D), be2.reshape(1, D)

    deg_fn = _make_deg(idx_rows)
    agg_fn = _make_agg(idx_rows)

    deg_p = deg_fn(dstp, zeros16, ones16)
    w0x = _mm_call(xp, W0)
    u, dinv = _scale_call(deg_p, w0x)
    p = agg_fn(srcp, dstp, u, zeros)
    y, s, q = _post_call(p, u, b0r, dinv)

    u = _mid_call(y, s, q, g0r, be0r, W1, dinv)
    p = agg_fn(srcp, dstp, u, zeros)
    y, s, q = _post_call(p, u, b1r, dinv)

    u = _mid_call(y, s, q, g1r, be1r, W2, dinv)
    p = agg_fn(srcp, dstp, u, zeros)
    y, s, q = _post_call(p, u, b2r, dinv)

    sums, cnt = _pool_call(y, s, q, g2r, be2r, batchp)
    cls, dom, feat = _heads_call(
        sums, cnt, Wc, bc.reshape(1, -1), Wd1, bd1.reshape(1, -1),
        Wd2, bd2.reshape(1, -1))
    return (cls, dom, feat)


# speedup vs baseline: 1.0810x; 1.0810x over previous
import functools

import jax
import jax.numpy as jnp
from jax import lax
from jax.experimental import pallas as pl
from jax.experimental.pallas import tpu as pltpu
from jax.experimental.pallas import tpu_sc as plsc

N = 10000
D = 128
G = 64
N_PAD = 10240
BLK = 512
NUM_BLK = N_PAD // BLK
NC = 2
NS = 16
NW = NC * NS
K = 4
SLAB = N_PAD // NS
DEGW = 128


def _sc_mesh():
    return plsc.VectorSubcoreMesh(core_axis_name="c", subcore_axis_name="s")


AGW = 128
AGK = 8


def _make_agg(idx_rows):
    rows_per_tile = idx_rows // NW
    groups = rows_per_tile // AGK

    @functools.partial(
        pl.kernel,
        out_type=jax.ShapeDtypeStruct((NC, N_PAD, D), jnp.float32),
        mesh=_sc_mesh(),
        scratch_types=[
            pltpu.VMEM((AGK, AGW), jnp.int32),
            pltpu.VMEM((AGK, AGW), jnp.int32),
            pltpu.VMEM((AGW, D), jnp.float32),
            pltpu.VMEM((AGW, D), jnp.float32),
            pltpu.VMEM_SHARED((N_PAD, D), jnp.float32),
            pltpu.SemaphoreType.DMA,
            pltpu.SemaphoreType.DMA,
            pltpu.SemaphoreType.DMA,
            pltpu.SemaphoreType.DMA,
        ],
    )
    def agg(src_hbm, dst_hbm, u_hbm, z_hbm, out_hbm, src_v, dst_v, rows0,
            rows1, acc, gsem0, gsem1, ssem0, ssem1):
        cid = lax.axis_index("c")
        sid = lax.axis_index("s")
        wid = sid * NC + cid
        shard = wid * rows_per_tile

        pltpu.sync_copy(z_hbm, rows0)
        for j in range(SLAB // AGW):
            pltpu.sync_copy(rows0, acc.at[pl.ds(sid * SLAB + j * AGW, AGW)])
        plsc.subcore_barrier()

        bufs = (rows0, rows1)
        gsems = (gsem0, gsem1)
        ssems = (ssem0, ssem1)

        def body(g, carry):
            base = shard + g * AGK
            pltpu.sync_copy(src_hbm.at[pl.ds(base, AGK)], src_v)
            pltpu.sync_copy(dst_hbm.at[pl.ds(base, AGK)], dst_v)
            scat = [None, None]
            gd = pltpu.async_copy(u_hbm.at[src_v.at[0]], bufs[0], gsems[0])
            for j in range(AGK):
                b = j % 2
                nb = (j + 1) % 2
                gnext = None
                if j + 1 < AGK:
                    if scat[nb] is not None:
                        scat[nb].wait()
                    gnext = pltpu.async_copy(
                        u_hbm.at[src_v.at[j + 1]], bufs[nb], gsems[nb])
                gd.wait()
                scat[b] = pltpu.async_copy(
                    bufs[b], acc.at[dst_v.at[j]], ssems[b], add=True)
                gd = gnext
            scat[(AGK - 2) % 2].wait()
            scat[(AGK - 1) % 2].wait()
            return carry

        lax.fori_loop(0, groups, body, 0)

        plsc.subcore_barrier()
        for j in range(SLAB // AGW):
            sl = pl.ds(sid * SLAB + j * AGW, AGW)
            pltpu.sync_copy(acc.at[sl], rows0)
            pltpu.sync_copy(rows0, out_hbm.at[cid, sl])

    return agg


def _make_deg(idx_rows):
    idx_rows2 = idx_rows * (128 // AGW)
    rows_per_tile = idx_rows2 // NW
    dgk = 2 * AGK
    groups = rows_per_tile // dgk

    @functools.partial(
        pl.kernel,
        out_type=jax.ShapeDtypeStruct((NC, N_PAD, DEGW), jnp.float32),
        mesh=_sc_mesh(),
        scratch_types=[
            pltpu.VMEM((dgk, AGW), jnp.int32),
            pltpu.VMEM((AGW, DEGW), jnp.float32),
            pltpu.VMEM_SHARED((N_PAD, DEGW), jnp.float32),
            pltpu.SemaphoreType.DMA,
        ],
    )
    def deg(dst_hbm, z_hbm, ones_hbm, out_hbm, dst_v, rows_v, acc, ssem):
        cid = lax.axis_index("c")
        sid = lax.axis_index("s")
        wid = sid * NC + cid

        pltpu.sync_copy(z_hbm, rows_v)
        for j in range(SLAB // AGW):
            pltpu.sync_copy(rows_v, acc.at[pl.ds(sid * SLAB + j * AGW, AGW)])
        pltpu.sync_copy(ones_hbm, rows_v)
        plsc.subcore_barrier()

        def body(g, carry):
            base = wid * rows_per_tile + g * dgk
            pltpu.sync_copy(dst_hbm.at[pl.ds(base, dgk)], dst_v)
            descs = [pltpu.async_copy(rows_v, acc.at[dst_v.at[j]], ssem,
                                      add=True)
                     for j in range(dgk)]
            for dsc in descs:
                dsc.wait()
            return carry

        lax.fori_loop(0, groups, body, 0)

        plsc.subcore_barrier()
        for j in range(SLAB // AGW):
            sl = pl.ds(sid * SLAB + j * AGW, AGW)
            pltpu.sync_copy(acc.at[sl], rows_v)
            pltpu.sync_copy(rows_v, out_hbm.at[cid, sl])

    return deg


def _rowmask(i):
    return i * BLK + lax.broadcasted_iota(jnp.int32, (BLK, 1), 0) < N


def _mm_body(x_ref, w_ref, o_ref):
    o_ref[...] = jnp.dot(x_ref[...], w_ref[...],
                         preferred_element_type=jnp.float32)


def _scale_body(dp_ref, wx_ref, u_ref, dinv_ref):
    i = pl.program_id(0)
    degsum = dp_ref[0, :, 0:1] + dp_ref[1, :, 0:1] + 1.0
    dinv = jnp.where(_rowmask(i), lax.rsqrt(degsum), 0.0)
    dinv_ref[...] = dinv
    u_ref[...] = wx_ref[...] * dinv


def _post_body(p_ref, u_ref, b_ref, dinv_ref, y_ref, s_ref, q_ref):
    i = pl.program_id(0)
    y = (p_ref[0] + p_ref[1] + u_ref[...]) * dinv_ref[...] + b_ref[...]
    y_ref[...] = y
    ym = jnp.where(_rowmask(i), y, 0.0)

    @pl.when(i == 0)
    def _():
        s_ref[...] = jnp.zeros_like(s_ref)
        q_ref[...] = jnp.zeros_like(q_ref)

    s_ref[0:1, :] = s_ref[0:1, :] + jnp.sum(ym, axis=0, keepdims=True)
    q_ref[0:1, :] = q_ref[0:1, :] + jnp.sum(ym * ym, axis=0, keepdims=True)


def _bn_relu(y, s_ref, q_ref, g_ref, be_ref):
    mu = s_ref[0:1, :] * (1.0 / N)
    var = q_ref[0:1, :] * (1.0 / N) - mu * mu
    scale = lax.rsqrt(var + 1e-5) * g_ref[...]
    return jnp.maximum((y - mu) * scale + be_ref[...], 0.0)


def _mid_body(y_ref, s_ref, q_ref, g_ref, be_ref, w_ref, dinv_ref, u_ref):
    h = _bn_relu(y_ref[...], s_ref, q_ref, g_ref, be_ref)
    u_ref[...] = (
        jnp.dot(h, w_ref[...], preferred_element_type=jnp.float32)
        * dinv_ref[...])


def _pool_body(y_ref, s_ref, q_ref, g_ref, be_ref, batch_ref, sums_ref,
               cnt_ref):
    i = pl.program_id(0)
    h = _bn_relu(y_ref[...], s_ref, q_ref, g_ref, be_ref)

    @pl.when(i == 0)
    def _():
        sums_ref[...] = jnp.zeros_like(sums_ref)
        cnt_ref[...] = jnp.zeros_like(cnt_ref)

    gids = lax.broadcasted_iota(jnp.int32, (G, 128), 0)
    sums = sums_ref[...]
    cnt = cnt_ref[...]
    for half in range(BLK // 128):
        b = batch_ref[0, half:half + 1, :]
        onehot = (gids == b).astype(jnp.float32)
        sums = sums + jnp.dot(onehot, h[half * 128:(half + 1) * 128, :],
                              preferred_element_type=jnp.float32)
        cnt = cnt + jnp.broadcast_to(
            jnp.sum(onehot, axis=1, keepdims=True), (G, 128))
    sums_ref[...] = sums
    cnt_ref[...] = cnt


def _heads_body(sums_ref, cnt_ref, wc_ref, bc_ref, wd1_ref, bd1_ref, wd2_ref,
                bd2_ref, cls_ref, dom_ref, feat_ref):
    cnt = jnp.maximum(cnt_ref[...], 1.0)
    feat = sums_ref[...] / cnt
    feat_ref[...] = feat
    cls_ref[...] = jnp.dot(
        feat, wc_ref[...], preferred_element_type=jnp.float32) + bc_ref[...]
    hd = jnp.maximum(
        jnp.dot(feat, wd1_ref[...], preferred_element_type=jnp.float32)
        + bd1_ref[...], 0.0)
    dom_ref[...] = jnp.dot(
        hd, wd2_ref[...], preferred_element_type=jnp.float32) + bd2_ref[...]


def _blk(i):
    return (i, 0)


def _const(i):
    return (0, 0)


_ROWSPEC = pl.BlockSpec((BLK, 128), _blk)
_COLSPEC = pl.BlockSpec((BLK, 1), _blk)
_FULLW = pl.BlockSpec((128, 128), _const)
_VEC = pl.BlockSpec((1, 128), _const)
_STAT = pl.BlockSpec((8, 128), _const)
_PSPEC = pl.BlockSpec((NC, BLK, 128), lambda i: (0, i, 0))


def _mm_call(xp, w):
    return pl.pallas_call(
        _mm_body,
        grid=(NUM_BLK,),
        in_specs=[_ROWSPEC, _FULLW],
        out_specs=_ROWSPEC,
        out_shape=jax.ShapeDtypeStruct((N_PAD, D), jnp.float32),
    )(xp, w)


def _scale_call(deg_p, wx):
    return pl.pallas_call(
        _scale_body,
        grid=(NUM_BLK,),
        in_specs=[pl.BlockSpec((NC, BLK, DEGW), lambda i: (0, i, 0)),
                  _ROWSPEC],
        out_specs=[_ROWSPEC, _COLSPEC],
        out_shape=[jax.ShapeDtypeStruct((N_PAD, D), jnp.float32),
                   jax.ShapeDtypeStruct((N_PAD, 1), jnp.float32)],
    )(deg_p, wx)


def _post_call(p, u, b, dinv):
    return pl.pallas_call(
        _post_body,
        grid=(NUM_BLK,),
        in_specs=[_PSPEC, _ROWSPEC, _VEC, _COLSPEC],
        out_specs=[_ROWSPEC, _STAT, _STAT],
        out_shape=[jax.ShapeDtypeStruct((N_PAD, D), jnp.float32),
                   jax.ShapeDtypeStruct((8, 128), jnp.float32),
                   jax.ShapeDtypeStruct((8, 128), jnp.float32)],
    )(p, u, b, dinv)


def _mid_call(y, s, q, g, be, w, dinv):
    return pl.pallas_call(
        _mid_body,
        grid=(NUM_BLK,),
        in_specs=[_ROWSPEC, _STAT, _STAT, _VEC, _VEC, _FULLW, _COLSPEC],
        out_specs=_ROWSPEC,
        out_shape=jax.ShapeDtypeStruct((N_PAD, D), jnp.float32),
    )(y, s, q, g, be, w, dinv)


def _pool_call(y, s, q, g, be, batch3d):
    return pl.pallas_call(
        _pool_body,
        grid=(NUM_BLK,),
        in_specs=[_ROWSPEC, _STAT, _STAT, _VEC, _VEC,
                  pl.BlockSpec((1, BLK // 128, 128), lambda i: (i, 0, 0))],
        out_specs=[pl.BlockSpec((G, 128), _const),
                   pl.BlockSpec((G, 128), _const)],
        out_shape=[jax.ShapeDtypeStruct((G, 128), jnp.float32),
                   jax.ShapeDtypeStruct((G, 128), jnp.float32)],
    )(y, s, q, g, be, batch3d)


def _heads_call(sums, cnt, wc, bc, wd1, bd1, wd2, bd2):
    return pl.pallas_call(
        _heads_body,
        out_shape=[jax.ShapeDtypeStruct((G, wc.shape[1]), jnp.float32),
                   jax.ShapeDtypeStruct((G, 2), jnp.float32),
                   jax.ShapeDtypeStruct((G, 128), jnp.float32)],
    )(sums, cnt, wc, bc, wd1, bd1, wd2, bd2)


def kernel(x, edge_index, batch, W0, b0, g0, be0, W1, b1, g1, be1,
           W2, b2, g2, be2, Wc, bc, Wd1, bd1, Wd2, bd2):
    e = edge_index.shape[1]
    gran = AGW * 2 * AGK * NW
    e_pad = -(-e // gran) * gran
    idx_rows = e_pad // 128
    npad = N_PAD - N

    src = edge_index[0].astype(jnp.int32)
    dst = edge_index[1].astype(jnp.int32)
    pad = (N + jnp.arange(e_pad - e, dtype=jnp.int32) % npad)
    srcp = jnp.concatenate([src, pad]).reshape(-1, AGW)
    dstp = jnp.concatenate([dst, pad]).reshape(-1, AGW)

    xp = jnp.zeros((N_PAD, D), jnp.float32).at[:N].set(x)
    batchp = jnp.concatenate(
        [batch.astype(jnp.int32),
         jnp.full((npad,), G + 63, jnp.int32)]).reshape(NUM_BLK, BLK // 128,
                                                        128)
    zeros = jnp.zeros((AGW, D), jnp.float32)
    zeros16 = jnp.zeros((AGW, DEGW), jnp.float32)
    ones16 = jnp.ones((AGW, DEGW), jnp.float32)

    b0r, g0r, be0r = b0.reshape(1, D), g0.reshape(1, D), be0.reshape(1, D)
    b1r, g1r, be1r = b1.reshape(1, D), g1.reshape(1, D), be1.reshape(1, D)
    b2r, g2r, be2r = b2.reshape(1, D), g2.reshape(1, D), be2.reshape(1, D)

    deg_fn = _make_deg(idx_rows)
    agg_fn = _make_agg(idx_rows)

    deg_p = deg_fn(dstp, zeros16, ones16)
    w0x = _mm_call(xp, W0)
    u, dinv = _scale_call(deg_p, w0x)
    p = agg_fn(srcp, dstp, u, zeros)
    y, s, q = _post_call(p, u, b0r, dinv)

    u = _mid_call(y, s, q, g0r, be0r, W1, dinv)
    p = agg_fn(srcp, dstp, u, zeros)
    y, s, q = _post_call(p, u, b1r, dinv)

    u = _mid_call(y, s, q, g1r, be1r, W2, dinv)
    p = agg_fn(srcp, dstp, u, zeros)
    y, s, q = _post_call(p, u, b2r, dinv)

    sums, cnt = _pool_call(y, s, q, g2r, be2r, batchp)
    cls, dom, feat = _heads_call(
        sums, cnt, Wc, bc.reshape(1, -1), Wd1, bd1.reshape(1, -1),
        Wd2, bd2.reshape(1, -1))
    return (cls, dom, feat)


# speedup vs baseline: 1.1505x; 1.0643x over previous
import functools

import jax
import jax.numpy as jnp
from jax import lax
from jax.experimental import pallas as pl
from jax.experimental.pallas import tpu as pltpu
from jax.experimental.pallas import tpu_sc as plsc

N = 10000
D = 128
G = 64
N_PAD = 10240
BLK = 1024
NUM_BLK = N_PAD // BLK
NC = 2
NS = 16
NW = NC * NS
SLAB = N_PAD // NS
DEGW = 128


def _sc_mesh():
    return plsc.VectorSubcoreMesh(core_axis_name="c", subcore_axis_name="s")


AGW = 128
AGK = 8


def _make_agg(idx_rows):
    rows_per_tile = idx_rows // NW
    groups = rows_per_tile // AGK

    @functools.partial(
        pl.kernel,
        out_type=jax.ShapeDtypeStruct((NC, N_PAD, D), jnp.float32),
        mesh=_sc_mesh(),
        scratch_types=[
            pltpu.VMEM((AGK, AGW), jnp.int32),
            pltpu.VMEM((AGK, AGW), jnp.int32),
            pltpu.VMEM((AGW, D), jnp.float32),
            pltpu.VMEM((AGW, D), jnp.float32),
            pltpu.VMEM_SHARED((N_PAD, D), jnp.float32),
            pltpu.SemaphoreType.DMA,
            pltpu.SemaphoreType.DMA,
            pltpu.SemaphoreType.DMA,
            pltpu.SemaphoreType.DMA,
        ],
    )
    def agg(src_hbm, dst_hbm, u_hbm, z_hbm, out_hbm, src_v, dst_v, rows0,
            rows1, acc, gsem0, gsem1, ssem0, ssem1):
        cid = lax.axis_index("c")
        sid = lax.axis_index("s")
        wid = sid * NC + cid
        shard = wid * rows_per_tile

        pltpu.sync_copy(z_hbm, rows0)
        for j in range(SLAB // AGW):
            pltpu.sync_copy(rows0, acc.at[pl.ds(sid * SLAB + j * AGW, AGW)])
        plsc.subcore_barrier()

        bufs = (rows0, rows1)
        gsems = (gsem0, gsem1)
        ssems = (ssem0, ssem1)

        def body(g, carry):
            base = shard + g * AGK
            pltpu.sync_copy(src_hbm.at[pl.ds(base, AGK)], src_v)
            pltpu.sync_copy(dst_hbm.at[pl.ds(base, AGK)], dst_v)
            scat = [None, None]
            gd = pltpu.async_copy(u_hbm.at[src_v.at[0]], bufs[0], gsems[0])
            for j in range(AGK):
                b = j % 2
                nb = (j + 1) % 2
                gnext = None
                if j + 1 < AGK:
                    if scat[nb] is not None:
                        scat[nb].wait()
                    gnext = pltpu.async_copy(
                        u_hbm.at[src_v.at[j + 1]], bufs[nb], gsems[nb])
                gd.wait()
                scat[b] = pltpu.async_copy(
                    bufs[b], acc.at[dst_v.at[j]], ssems[b], add=True)
                gd = gnext
            scat[(AGK - 2) % 2].wait()
            scat[(AGK - 1) % 2].wait()
            return carry

        lax.fori_loop(0, groups, body, 0)

        plsc.subcore_barrier()
        for j in range(SLAB // AGW):
            sl = pl.ds(sid * SLAB + j * AGW, AGW)
            pltpu.sync_copy(acc.at[sl], rows0)
            pltpu.sync_copy(rows0, out_hbm.at[cid, sl])

    return agg


def _make_deg(idx_rows):
    idx_rows2 = idx_rows * (128 // AGW)
    rows_per_tile = idx_rows2 // NW
    dgk = 2 * AGK
    groups = rows_per_tile // dgk

    @functools.partial(
        pl.kernel,
        out_type=jax.ShapeDtypeStruct((NC, N_PAD, DEGW), jnp.float32),
        mesh=_sc_mesh(),
        scratch_types=[
            pltpu.VMEM((dgk, AGW), jnp.int32),
            pltpu.VMEM((AGW, DEGW), jnp.float32),
            pltpu.VMEM_SHARED((N_PAD, DEGW), jnp.float32),
            pltpu.SemaphoreType.DMA,
        ],
    )
    def deg(dst_hbm, z_hbm, ones_hbm, out_hbm, dst_v, rows_v, acc, ssem):
        cid = lax.axis_index("c")
        sid = lax.axis_index("s")
        wid = sid * NC + cid

        pltpu.sync_copy(z_hbm, rows_v)
        for j in range(SLAB // AGW):
            pltpu.sync_copy(rows_v, acc.at[pl.ds(sid * SLAB + j * AGW, AGW)])
        pltpu.sync_copy(ones_hbm, rows_v)
        plsc.subcore_barrier()

        def body(g, carry):
            base = wid * rows_per_tile + g * dgk
            pltpu.sync_copy(dst_hbm.at[pl.ds(base, dgk)], dst_v)
            descs = [pltpu.async_copy(rows_v, acc.at[dst_v.at[j]], ssem,
                                      add=True)
                     for j in range(dgk)]
            for dsc in descs:
                dsc.wait()
            return carry

        lax.fori_loop(0, groups, body, 0)

        plsc.subcore_barrier()
        for j in range(SLAB // AGW):
            sl = pl.ds(sid * SLAB + j * AGW, AGW)
            pltpu.sync_copy(acc.at[sl], rows_v)
            pltpu.sync_copy(rows_v, out_hbm.at[cid, sl])

    return deg


def _rowmask(i):
    return i * BLK + lax.broadcasted_iota(jnp.int32, (BLK, 1), 0) < N


def _mm_body(x_ref, w_ref, o_ref):
    o_ref[...] = jnp.dot(x_ref[...], w_ref[...],
                         preferred_element_type=jnp.float32)


def _scale_body(dp_ref, wx_ref, u_ref, dinv_ref):
    i = pl.program_id(0)
    degsum = dp_ref[0, :, 0:1] + dp_ref[1, :, 0:1] + 1.0
    dinv = jnp.where(_rowmask(i), lax.rsqrt(degsum), 0.0)
    dinv_ref[...] = dinv
    u_ref[...] = wx_ref[...] * dinv


def _post_body(p_ref, u_ref, b_ref, dinv_ref, y_ref, s_ref, q_ref):
    i = pl.program_id(0)
    y = (p_ref[0] + p_ref[1] + u_ref[...]) * dinv_ref[...] + b_ref[...]
    y_ref[...] = y
    ym = jnp.where(_rowmask(i), y, 0.0)

    @pl.when(i == 0)
    def _():
        s_ref[...] = jnp.zeros_like(s_ref)
        q_ref[...] = jnp.zeros_like(q_ref)

    s_ref[0:1, :] = s_ref[0:1, :] + jnp.sum(ym, axis=0, keepdims=True)
    q_ref[0:1, :] = q_ref[0:1, :] + jnp.sum(ym * ym, axis=0, keepdims=True)


def _bn_relu(y, s_ref, q_ref, g_ref, be_ref):
    mu = s_ref[0:1, :] * (1.0 / N)
    var = q_ref[0:1, :] * (1.0 / N) - mu * mu
    scale = lax.rsqrt(var + 1e-5) * g_ref[...]
    return jnp.maximum((y - mu) * scale + be_ref[...], 0.0)


def _mid_body(y_ref, s_ref, q_ref, g_ref, be_ref, w_ref, dinv_ref, u_ref):
    h = _bn_relu(y_ref[...], s_ref, q_ref, g_ref, be_ref)
    u_ref[...] = (
        jnp.dot(h, w_ref[...], preferred_element_type=jnp.float32)
        * dinv_ref[...])


def _pool_body(y_ref, s_ref, q_ref, g_ref, be_ref, batch_ref, sums_ref,
               cnt_ref):
    i = pl.program_id(0)
    h = _bn_relu(y_ref[...], s_ref, q_ref, g_ref, be_ref)

    @pl.when(i == 0)
    def _():
        sums_ref[...] = jnp.zeros_like(sums_ref)
        cnt_ref[...] = jnp.zeros_like(cnt_ref)

    gids = lax.broadcasted_iota(jnp.int32, (G, 128), 0)
    sums = sums_ref[...]
    cnt = cnt_ref[...]
    for half in range(BLK // 128):
        b = batch_ref[0, half:half + 1, :]
        onehot = (gids == b).astype(jnp.float32)
        sums = sums + jnp.dot(onehot, h[half * 128:(half + 1) * 128, :],
                              preferred_element_type=jnp.float32)
        cnt = cnt + jnp.broadcast_to(
            jnp.sum(onehot, axis=1, keepdims=True), (G, 128))
    sums_ref[...] = sums
    cnt_ref[...] = cnt


def _heads_body(sums_ref, cnt_ref, wc_ref, bc_ref, wd1_ref, bd1_ref, wd2_ref,
                bd2_ref, cls_ref, dom_ref, feat_ref):
    cnt = jnp.maximum(cnt_ref[...], 1.0)
    feat = sums_ref[...] / cnt
    feat_ref[...] = feat
    cls_ref[...] = jnp.dot(
        feat, wc_ref[...], preferred_element_type=jnp.float32) + bc_ref[...]
    hd = jnp.maximum(
        jnp.dot(feat, wd1_ref[...], preferred_element_type=jnp.float32)
        + bd1_ref[...], 0.0)
    dom_ref[...] = jnp.dot(
        hd, wd2_ref[...], preferred_element_type=jnp.float32) + bd2_ref[...]


def _blk(i):
    return (i, 0)


def _const(i):
    return (0, 0)


_ROWSPEC = pl.BlockSpec((BLK, 128), _blk)
_COLSPEC = pl.BlockSpec((BLK, 1), _blk)
_FULLW = pl.BlockSpec((128, 128), _const)
_VEC = pl.BlockSpec((1, 128), _const)
_STAT = pl.BlockSpec((8, 128), _const)
_PSPEC = pl.BlockSpec((NC, BLK, 128), lambda i: (0, i, 0))


def _mm_call(xp, w):
    return pl.pallas_call(
        _mm_body,
        grid=(NUM_BLK,),
        in_specs=[_ROWSPEC, _FULLW],
        out_specs=_ROWSPEC,
        out_shape=jax.ShapeDtypeStruct((N_PAD, D), jnp.float32),
    )(xp, w)


def _scale_call(deg_p, wx):
    return pl.pallas_call(
        _scale_body,
        grid=(NUM_BLK,),
        in_specs=[pl.BlockSpec((NC, BLK, DEGW), lambda i: (0, i, 0)),
                  _ROWSPEC],
        out_specs=[_ROWSPEC, _COLSPEC],
        out_shape=[jax.ShapeDtypeStruct((N_PAD, D), jnp.float32),
                   jax.ShapeDtypeStruct((N_PAD, 1), jnp.float32)],
    )(deg_p, wx)


def _post_call(p, u, b, dinv):
    return pl.pallas_call(
        _post_body,
        grid=(NUM_BLK,),
        in_specs=[_PSPEC, _ROWSPEC, _VEC, _COLSPEC],
        out_specs=[_ROWSPEC, _STAT, _STAT],
        out_shape=[jax.ShapeDtypeStruct((N_PAD, D), jnp.float32),
                   jax.ShapeDtypeStruct((8, 128), jnp.float32),
                   jax.ShapeDtypeStruct((8, 128), jnp.float32)],
    )(p, u, b, dinv)


def _mid_call(y, s, q, g, be, w, dinv):
    return pl.pallas_call(
        _mid_body,
        grid=(NUM_BLK,),
        in_specs=[_ROWSPEC, _STAT, _STAT, _VEC, _VEC, _FULLW, _COLSPEC],
        out_specs=_ROWSPEC,
        out_shape=jax.ShapeDtypeStruct((N_PAD, D), jnp.float32),
    )(y, s, q, g, be, w, dinv)


def _pool_call(y, s, q, g, be, batch3d):
    return pl.pallas_call(
        _pool_body,
        grid=(NUM_BLK,),
        in_specs=[_ROWSPEC, _STAT, _STAT, _VEC, _VEC,
                  pl.BlockSpec((1, BLK // 128, 128), lambda i: (i, 0, 0))],
        out_specs=[pl.BlockSpec((G, 128), _const),
                   pl.BlockSpec((G, 128), _const)],
        out_shape=[jax.ShapeDtypeStruct((G, 128), jnp.float32),
                   jax.ShapeDtypeStruct((G, 128), jnp.float32)],
    )(y, s, q, g, be, batch3d)


def _heads_call(sums, cnt, wc, bc, wd1, bd1, wd2, bd2):
    return pl.pallas_call(
        _heads_body,
        out_shape=[jax.ShapeDtypeStruct((G, wc.shape[1]), jnp.float32),
                   jax.ShapeDtypeStruct((G, 2), jnp.float32),
                   jax.ShapeDtypeStruct((G, 128), jnp.float32)],
    )(sums, cnt, wc, bc, wd1, bd1, wd2, bd2)


def kernel(x, edge_index, batch, W0, b0, g0, be0, W1, b1, g1, be1,
           W2, b2, g2, be2, Wc, bc, Wd1, bd1, Wd2, bd2):
    e = edge_index.shape[1]
    gran = AGW * 2 * AGK * NW
    e_pad = -(-e // gran) * gran
    idx_rows = e_pad // 128
    npad = N_PAD - N

    src = edge_index[0].astype(jnp.int32)
    dst = edge_index[1].astype(jnp.int32)
    pad = (N + jnp.arange(e_pad - e, dtype=jnp.int32) % npad)
    srcp = jnp.concatenate([src, pad]).reshape(-1, AGW)
    dstp = jnp.concatenate([dst, pad]).reshape(-1, AGW)

    xp = jnp.zeros((N_PAD, D), jnp.float32).at[:N].set(x)
    batchp = jnp.concatenate(
        [batch.astype(jnp.int32),
         jnp.full((npad,), G + 63, jnp.int32)]).reshape(NUM_BLK, BLK // 128,
                                                        128)
    zeros = jnp.zeros((AGW, D), jnp.float32)
    zeros16 = jnp.zeros((AGW, DEGW), jnp.float32)
    ones16 = jnp.ones((AGW, DEGW), jnp.float32)

    b0r, g0r, be0r = b0.reshape(1, D), g0.reshape(1, D), be0.reshape(1, D)
    b1r, g1r, be1r = b1.reshape(1, D), g1.reshape(1, D), be1.reshape(1, D)
    b2r, g2r, be2r = b2.reshape(1, D), g2.reshape(1, D), be2.reshape(1, D)

    deg_fn = _make_deg(idx_rows)
    agg_fn = _make_agg(idx_rows)

    deg_p = deg_fn(dstp, zeros16, ones16)
    w0x = _mm_call(xp, W0)
    u, dinv = _scale_call(deg_p, w0x)
    p = agg_fn(srcp, dstp, u, zeros)
    y, s, q = _post_call(p, u, b0r, dinv)

    u = _mid_call(y, s, q, g0r, be0r, W1, dinv)
    p = agg_fn(srcp, dstp, u, zeros)
    y, s, q = _post_call(p, u, b1r, dinv)

    u = _mid_call(y, s, q, g1r, be1r, W2, dinv)
    p = agg_fn(srcp, dstp, u, zeros)
    y, s, q = _post_call(p, u, b2r, dinv)

    sums, cnt = _pool_call(y, s, q, g2r, be2r, batchp)
    cls, dom, feat = _heads_call(
        sums, cnt, Wc, bc.reshape(1, -1), Wd1, bd1.reshape(1, -1),
        Wd2, bd2.reshape(1, -1))
    return (cls, dom, feat)
